# baseline (speedup 1.0000x reference)
import functools

import jax
import jax.numpy as jnp
from jax.experimental import pallas as pl
from jax.experimental.pallas import tpu as pltpu

TN = 256


def _fused_kernel(B, HD, nblk, lower_ref, xT_ref, Wtop_ref, Wbot_ref,
                  b1t_ref, BW20_ref, BW21_ref, b2t_ref,
                  We_ref, be_ref, Wo_ref, bo_ref, out_ref,
                  L16_ref, h_ref, m_ref):
    i = pl.program_id(0)

    @pl.when(i < nblk)
    def _phase1():
        L16 = lower_ref[...].astype(jnp.bfloat16)
        L16_ref[pl.ds(i * TN, TN), :] = L16
        xT16 = xT_ref[...].astype(jnp.bfloat16)
        s = jnp.dot(L16, xT16, preferred_element_type=jnp.float32)
        xr = xT_ref[pl.ds(i * TN, TN), :]
        hb = (jnp.dot(xr, Wtop_ref[...], preferred_element_type=jnp.float32)
              + jnp.dot(s, Wbot_ref[...], preferred_element_type=jnp.float32)
              + b1t_ref[...])
        h_ref[pl.ds(i * TN, TN), :] = jnp.maximum(hb, 0.0).astype(jnp.bfloat16)

    @pl.when(i >= nblk)
    def _phase2():
        j = i - nblk
        Lb = L16_ref[pl.ds(j * TN, TN), :]
        agg16 = jnp.dot(Lb, h_ref[...],
                        preferred_element_type=jnp.float32
                        ).astype(jnp.bfloat16)
        Hi = h_ref[pl.ds(j * TN, TN), :]
        G = (jnp.dot(Hi, BW20_ref[...], preferred_element_type=jnp.float32)
             + jnp.dot(agg16, BW21_ref[...], preferred_element_type=jnp.float32)
             + b2t_ref[...])
        G = jnp.maximum(G, 0.0)
        Gm = jnp.max(G, axis=0, keepdims=True)

        @pl.when(j == 0)
        def _():
            m_ref[...] = Gm

        @pl.when(j > 0)
        def _():
            m_ref[...] = jnp.maximum(m_ref[...], Gm)

        @pl.when(j == nblk - 1)
        def _():
            mm = m_ref[...].reshape(B, HD)
            t = jnp.dot(mm, We_ref[...], preferred_element_type=jnp.float32)
            t = jnp.maximum(t + be_ref[...], 0.0)
            out_ref[...] = (jnp.dot(t, Wo_ref[...],
                                    preferred_element_type=jnp.float32)
                            + bo_ref[...])


def kernel(x, lower, _, W1_0, W1_1, b1, W2_0, W2_1, b2, We, be, Wo, bo):
    B, N, _d = x.shape
    HD = W1_0.shape[1]
    HFF = We.shape[1]
    NC = Wo.shape[1]
    nblk = N // TN

    xT = x[:, :, 0].T
    eyeB = jnp.eye(B, dtype=jnp.float32)
    Wtop = jnp.kron(eyeB, W1_0)
    Wbot = jnp.kron(eyeB, W1_1)
    b1t = jnp.tile(b1, B).reshape(1, B * HD)
    BW20 = jnp.kron(eyeB, W2_0).astype(jnp.bfloat16)
    BW21 = jnp.kron(eyeB, W2_1).astype(jnp.bfloat16)
    b2t = jnp.tile(b2, B).reshape(1, B * HD)
    ber = be.reshape(1, HFF)
    bor = bo.reshape(1, NC)

    cidx = lambda i: (0, 0)
    out = pl.pallas_call(
        functools.partial(_fused_kernel, B, HD, nblk),
        grid=(2 * nblk,),
        in_specs=[
            pl.BlockSpec((TN, N), lambda i: (jnp.minimum(i, nblk - 1), 0)),
            pl.BlockSpec((N, B), cidx),
            pl.BlockSpec((B, B * HD), cidx),
            pl.BlockSpec((B, B * HD), cidx),
            pl.BlockSpec((1, B * HD), cidx),
            pl.BlockSpec((B * HD, B * HD), cidx),
            pl.BlockSpec((B * HD, B * HD), cidx),
            pl.BlockSpec((1, B * HD), cidx),
            pl.BlockSpec((HD, HFF), cidx),
            pl.BlockSpec((1, HFF), cidx),
            pl.BlockSpec((HFF, NC), cidx),
            pl.BlockSpec((1, NC), cidx),
        ],
        out_specs=pl.BlockSpec((B, NC), cidx),
        out_shape=jax.ShapeDtypeStruct((B, NC), jnp.float32),
        scratch_shapes=[
            pltpu.VMEM((N, N), jnp.bfloat16),
            pltpu.VMEM((N, B * HD), jnp.bfloat16),
            pltpu.VMEM((1, B * HD), jnp.float32),
        ],
    )(lower, xT, Wtop, Wbot, b1t, BW20, BW21, b2t, We, ber, Wo, bor)

    return out
